# baseline (speedup 1.0000x reference)
import functools
import jax
import jax.numpy as jnp
from jax import lax
from jax.experimental import pallas as pl
from jax.experimental.pallas import tpu as pltpu
from jax.experimental.pallas import tpu_sc as plsc

_BT = 512
_WT = 128
_SL = 16


def _logits_kernel(x_ref, gw_ref, lt_ref):
    logits = lax.dot_general(x_ref[:], gw_ref[:], (((1,), (1,)), ((), ())),
                             preferred_element_type=jnp.float32)
    lt_ref[:] = lax.transpose(logits, (1, 0))


def _sc_route(l_hbm, w_hbm, cnt_hbm, z_hbm, lv, wv, cntv, zb, sem, semz):
    n_exp = l_hbm.shape[0]
    hidden = z_hbm.shape[1]
    wid = lax.axis_index("s") * 2 + lax.axis_index("c")
    base = wid * _WT
    lcps = [pltpu.make_async_copy(l_hbm.at[e, pl.ds(base, _WT)], lv.at[e],
                                  sem) for e in range(n_exp)]
    for cp in lcps:
        cp.start()
    nzr = zb.shape[0]
    for r in range(nzr):
        zb[r, :] = jnp.zeros((hidden,), jnp.float32)
    zcps = [pltpu.make_async_copy(
        zb, z_hbm.at[pl.ds(base + k * nzr, nzr), :], semz)
        for k in range(_WT // nzr)]
    for cp in zcps:
        cp.start()
    for cp in lcps:
        cp.wait()

    accs = [jnp.zeros((_SL,), jnp.int32) for _ in range(n_exp)]
    for j in range(_WT // _SL):
        sl = pl.ds(j * _SL, _SL)
        ls = [lv[e, sl] for e in range(n_exp)]
        m = ls[0]
        for e in range(1, n_exp):
            m = jnp.maximum(m, ls[e])
        zs = [jnp.exp(l - m) for l in ls]
        s = zs[0]
        for e in range(1, n_exp):
            s = s + zs[e]
        ps = [z / s for z in zs]
        m1 = ps[0]
        for e in range(1, n_exp):
            m1 = jnp.maximum(m1, ps[e])
        i1 = jnp.full((_SL,), n_exp, jnp.int32)
        for e in range(n_exp):
            i1 = jnp.minimum(i1, jnp.where(ps[e] == m1, e, n_exp))
        p2 = [jnp.where(i1 == e, -1.0, ps[e]) for e in range(n_exp)]
        m2 = p2[0]
        for e in range(1, n_exp):
            m2 = jnp.maximum(m2, p2[e])
        i2 = jnp.full((_SL,), n_exp, jnp.int32)
        for e in range(n_exp):
            i2 = jnp.minimum(i2, jnp.where(p2[e] == m2, e, n_exp))
        for e in range(n_exp):
            hit = jnp.logical_or(i1 == e, i2 == e)
            wv[e, sl] = jnp.where(hit, ps[e], 0.0)
            accs[e] = accs[e] + jnp.where(hit, 1, 0).astype(jnp.int32)

    for e in range(n_exp):
        cntv[pl.ds(e * _SL, _SL)] = accs[e]
    ccp = pltpu.make_async_copy(cntv, cnt_hbm.at[wid], sem)
    wcps = [pltpu.make_async_copy(wv.at[e], w_hbm.at[e, pl.ds(base, _WT)],
                                  sem) for e in range(n_exp)]
    ccp.start()
    for cp in wcps:
        cp.start()
    ccp.wait()
    for cp in wcps:
        cp.wait()
    for cp in zcps:
        cp.wait()


def _ffn_kernel(cnt_ref, outz_hbm, x_hbm, wt_hbm, w1_hbm, b1_hbm, w2_hbm,
                b2_hbm, out_hbm, flags_ref, acc, xch, wch, w1s, w2s, b1s,
                b2s, semx, semw, sema, sem1, sem2, sem3, sem4):
    n_tok, hidden = x_hbm.shape
    n_exp = w1_hbm.shape[0]
    cap = int(1.25 * n_tok / n_exp)
    nb = n_tok // _BT
    del outz_hbm

    cnt = cnt_ref[:]
    any_active = jnp.int32(0)
    for e in range(n_exp):
        c_e = jnp.sum(cnt[:, e * _SL:(e + 1) * _SL])
        f_e = jnp.logical_and(c_e > 0, c_e <= cap).astype(jnp.int32)
        flags_ref[e] = f_e
        any_active = jnp.maximum(any_active, f_e)
    flags_ref[n_exp] = any_active

    @pl.when(any_active != 0)
    def _():
        acc[:] = jnp.zeros_like(acc)

        def expert_body(e, carry):
            @pl.when(flags_ref[e] != 0)
            def _():
                cp1 = pltpu.make_async_copy(w1_hbm.at[e], w1s, sem1)
                cp2 = pltpu.make_async_copy(w2_hbm.at[e], w2s, sem2)
                cp3 = pltpu.make_async_copy(b1_hbm.at[e], b1s, sem3)
                cp4 = pltpu.make_async_copy(b2_hbm.at[e], b2s, sem4)
                cp1.start(); cp2.start(); cp3.start(); cp4.start()
                cp1.wait(); cp2.wait(); cp3.wait(); cp4.wait()

                def blk_body(b, carry2):
                    ds = pl.ds(b * _BT, _BT)
                    cpx = pltpu.make_async_copy(x_hbm.at[ds, :], xch, semx)
                    cpw = pltpu.make_async_copy(wt_hbm.at[:, ds], wch, semw)
                    cpx.start(); cpw.start()
                    cpx.wait(); cpw.wait()
                    h = lax.dot_general(
                        xch[:], w1s[:], (((1,), (1,)), ((), ())),
                        preferred_element_type=jnp.float32)
                    h = jnp.maximum(h + b1s[:], 0.0)
                    oe = lax.dot_general(
                        h, w2s[:], (((1,), (1,)), ((), ())),
                        preferred_element_type=jnp.float32)
                    oe = oe + b2s[:]
                    wt = lax.transpose(wch[:], (1, 0))
                    le = lax.broadcasted_iota(jnp.int32, wt.shape, 1)
                    wcol = jnp.sum(jnp.where(le == e, wt, 0.0),
                                   axis=1, keepdims=True)
                    acc[ds, :] += oe * wcol
                    return carry2

                lax.fori_loop(0, nb, blk_body, 0)
            return carry

        lax.fori_loop(0, n_exp, expert_body, 0)

        def wb_body(b, carry):
            ds = pl.ds(b * _BT, _BT)
            cpo = pltpu.make_async_copy(acc.at[ds, :], out_hbm.at[ds, :],
                                        sema)
            cpo.start()
            cpo.wait()
            return carry

        lax.fori_loop(0, nb, wb_body, 0)


def kernel(tokens, gate_w, w1, b1, w2, b2):
    batch, seq, hidden = tokens.shape
    n_tok = batch * seq
    n_exp = gate_w.shape[0]
    cap = int(1.25 * n_tok / n_exp)
    x = tokens.reshape(n_tok, hidden)
    nb = n_tok // _BT
    nw = n_tok // _WT

    lt = pl.pallas_call(
        _logits_kernel,
        grid=(nb,),
        in_specs=[
            pl.BlockSpec((_BT, hidden), lambda i: (i, 0)),
            pl.BlockSpec((n_exp, hidden), lambda i: (0, 0)),
        ],
        out_specs=pl.BlockSpec((n_exp, _BT), lambda i: (0, i)),
        out_shape=jax.ShapeDtypeStruct((n_exp, n_tok), jnp.float32),
        compiler_params=pltpu.CompilerParams(
            dimension_semantics=("arbitrary",)),
    )(x, gate_w)

    mesh = plsc.VectorSubcoreMesh(core_axis_name="c", subcore_axis_name="s")
    wt, cnts, outz = pl.kernel(
        _sc_route,
        mesh=mesh,
        out_type=[
            jax.ShapeDtypeStruct((n_exp, n_tok), jnp.float32),
            jax.ShapeDtypeStruct((nw, n_exp * _SL), jnp.int32),
            jax.ShapeDtypeStruct((n_tok, hidden), jnp.float32),
        ],
        scratch_types=[
            pltpu.VMEM((n_exp, _WT), jnp.float32),
            pltpu.VMEM((n_exp, _WT), jnp.float32),
            pltpu.VMEM((n_exp * _SL,), jnp.int32),
            pltpu.VMEM((32, 1024), jnp.float32),
            pltpu.SemaphoreType.DMA,
            pltpu.SemaphoreType.DMA,
        ],
    )(lt)

    out = pl.pallas_call(
        _ffn_kernel,
        in_specs=[
            pl.BlockSpec(memory_space=pltpu.MemorySpace.VMEM),
            pl.BlockSpec(memory_space=pltpu.MemorySpace.HBM),
            pl.BlockSpec(memory_space=pltpu.MemorySpace.HBM),
            pl.BlockSpec(memory_space=pltpu.MemorySpace.HBM),
            pl.BlockSpec(memory_space=pltpu.MemorySpace.HBM),
            pl.BlockSpec(memory_space=pltpu.MemorySpace.HBM),
            pl.BlockSpec(memory_space=pltpu.MemorySpace.HBM),
            pl.BlockSpec(memory_space=pltpu.MemorySpace.HBM),
        ],
        out_specs=pl.BlockSpec(memory_space=pltpu.MemorySpace.HBM),
        out_shape=jax.ShapeDtypeStruct((n_tok, hidden), jnp.float32),
        input_output_aliases={1: 0},
        scratch_shapes=[
            pltpu.SMEM((n_exp + 1,), jnp.int32),
            pltpu.VMEM((n_tok, hidden), jnp.float32),
            pltpu.VMEM((_BT, hidden), jnp.float32),
            pltpu.VMEM((n_exp, _BT), jnp.float32),
            pltpu.VMEM((hidden, hidden), jnp.float32),
            pltpu.VMEM((hidden, hidden), jnp.float32),
            pltpu.VMEM((1, hidden), jnp.float32),
            pltpu.VMEM((1, hidden), jnp.float32),
            pltpu.SemaphoreType.DMA,
            pltpu.SemaphoreType.DMA,
            pltpu.SemaphoreType.DMA,
            pltpu.SemaphoreType.DMA,
            pltpu.SemaphoreType.DMA,
            pltpu.SemaphoreType.DMA,
            pltpu.SemaphoreType.DMA,
        ],
    )(cnts, outz, x, wt, w1, b1.reshape(n_exp, 1, hidden),
      w2, b2.reshape(n_exp, 1, hidden))

    return out.reshape(batch, seq, hidden)


# speedup vs baseline: 1.1194x; 1.1194x over previous
import functools
import jax
import jax.numpy as jnp
from jax import lax
from jax.experimental import pallas as pl
from jax.experimental.pallas import tpu as pltpu
from jax.experimental.pallas import tpu_sc as plsc

_BT = 512
_WT = 128
_SL = 16


def _logits_kernel(x_ref, gw_ref, lt_ref, out_hbm, zbuf, semz):
    n_tok = out_hbm.shape[0]
    nb = n_tok // _BT
    i = pl.program_id(0)

    @pl.when(i == 0)
    def _():
        zbuf[:] = jnp.zeros_like(zbuf)

    pltpu.make_async_copy(zbuf, out_hbm.at[pl.ds(i * _BT, _BT), :],
                          semz).start()

    logits = lax.dot_general(x_ref[:], gw_ref[:], (((1,), (1,)), ((), ())),
                             preferred_element_type=jnp.float32)
    lt_ref[:] = lax.transpose(logits, (1, 0))

    @pl.when(i == nb - 1)
    def _():
        for _b in range(nb):
            pltpu.make_async_copy(
                zbuf, out_hbm.at[pl.ds(0, _BT), :], semz).wait()


def _sc_route(l_hbm, w_hbm, cnt_hbm, lv, wv, cntv, sem):
    n_exp = l_hbm.shape[0]
    wid = lax.axis_index("s") * 2 + lax.axis_index("c")
    base = wid * _WT
    lcp = pltpu.make_async_copy(l_hbm.at[:, pl.ds(base, _WT)], lv, sem)
    lcp.start()
    lcp.wait()

    accs = [jnp.zeros((_SL,), jnp.int32) for _ in range(n_exp)]
    for j in range(_WT // _SL):
        sl = pl.ds(j * _SL, _SL)
        ls = [lv[e, sl] for e in range(n_exp)]
        m = ls[0]
        for e in range(1, n_exp):
            m = jnp.maximum(m, ls[e])
        zs = [jnp.exp(l - m) for l in ls]
        s = zs[0]
        for e in range(1, n_exp):
            s = s + zs[e]
        ps = [z / s for z in zs]
        m1 = ps[0]
        for e in range(1, n_exp):
            m1 = jnp.maximum(m1, ps[e])
        i1 = jnp.full((_SL,), n_exp, jnp.int32)
        for e in range(n_exp):
            i1 = jnp.minimum(i1, jnp.where(ps[e] == m1, e, n_exp))
        p2 = [jnp.where(i1 == e, -1.0, ps[e]) for e in range(n_exp)]
        m2 = p2[0]
        for e in range(1, n_exp):
            m2 = jnp.maximum(m2, p2[e])
        i2 = jnp.full((_SL,), n_exp, jnp.int32)
        for e in range(n_exp):
            i2 = jnp.minimum(i2, jnp.where(p2[e] == m2, e, n_exp))
        for e in range(n_exp):
            hit = jnp.logical_or(i1 == e, i2 == e)
            wv[e, sl] = jnp.where(hit, ps[e], 0.0)
            accs[e] = accs[e] + jnp.where(hit, 1, 0).astype(jnp.int32)

    for e in range(n_exp):
        cntv[pl.ds(e * _SL, _SL)] = accs[e]
    ccp = pltpu.make_async_copy(cntv, cnt_hbm.at[wid], sem)
    wcp = pltpu.make_async_copy(wv, w_hbm.at[:, pl.ds(base, _WT)], sem)
    ccp.start()
    wcp.start()
    ccp.wait()
    wcp.wait()


def _ffn_kernel(cnt_ref, outz_hbm, x_hbm, wt_hbm, w1_hbm, b1_hbm, w2_hbm,
                b2_hbm, out_hbm, flags_ref, acc, xch, wch, w1s, w2s, b1s,
                b2s, semx, semw, sema, sem1, sem2, sem3, sem4):
    n_tok, hidden = x_hbm.shape
    n_exp = w1_hbm.shape[0]
    cap = int(1.25 * n_tok / n_exp)
    nb = n_tok // _BT
    del outz_hbm

    cnt = cnt_ref[:]
    any_active = jnp.int32(0)
    for e in range(n_exp):
        c_e = jnp.sum(cnt[:, e * _SL:(e + 1) * _SL])
        f_e = jnp.logical_and(c_e > 0, c_e <= cap).astype(jnp.int32)
        flags_ref[e] = f_e
        any_active = jnp.maximum(any_active, f_e)
    flags_ref[n_exp] = any_active

    @pl.when(any_active != 0)
    def _():
        acc[:] = jnp.zeros_like(acc)

        def expert_body(e, carry):
            @pl.when(flags_ref[e] != 0)
            def _():
                cp1 = pltpu.make_async_copy(w1_hbm.at[e], w1s, sem1)
                cp2 = pltpu.make_async_copy(w2_hbm.at[e], w2s, sem2)
                cp3 = pltpu.make_async_copy(b1_hbm.at[e], b1s, sem3)
                cp4 = pltpu.make_async_copy(b2_hbm.at[e], b2s, sem4)
                cp1.start(); cp2.start(); cp3.start(); cp4.start()
                cp1.wait(); cp2.wait(); cp3.wait(); cp4.wait()

                def blk_body(b, carry2):
                    ds = pl.ds(b * _BT, _BT)
                    cpx = pltpu.make_async_copy(x_hbm.at[ds, :], xch, semx)
                    cpw = pltpu.make_async_copy(wt_hbm.at[:, ds], wch, semw)
                    cpx.start(); cpw.start()
                    cpx.wait(); cpw.wait()
                    h = lax.dot_general(
                        xch[:], w1s[:], (((1,), (1,)), ((), ())),
                        preferred_element_type=jnp.float32)
                    h = jnp.maximum(h + b1s[:], 0.0)
                    oe = lax.dot_general(
                        h, w2s[:], (((1,), (1,)), ((), ())),
                        preferred_element_type=jnp.float32)
                    oe = oe + b2s[:]
                    wt = lax.transpose(wch[:], (1, 0))
                    le = lax.broadcasted_iota(jnp.int32, wt.shape, 1)
                    wcol = jnp.sum(jnp.where(le == e, wt, 0.0),
                                   axis=1, keepdims=True)
                    acc[ds, :] += oe * wcol
                    return carry2

                lax.fori_loop(0, nb, blk_body, 0)
            return carry

        lax.fori_loop(0, n_exp, expert_body, 0)

        def wb_body(b, carry):
            ds = pl.ds(b * _BT, _BT)
            cpo = pltpu.make_async_copy(acc.at[ds, :], out_hbm.at[ds, :],
                                        sema)
            cpo.start()
            cpo.wait()
            return carry

        lax.fori_loop(0, nb, wb_body, 0)


def kernel(tokens, gate_w, w1, b1, w2, b2):
    batch, seq, hidden = tokens.shape
    n_tok = batch * seq
    n_exp = gate_w.shape[0]
    cap = int(1.25 * n_tok / n_exp)
    x = tokens.reshape(n_tok, hidden)
    nb = n_tok // _BT
    nw = n_tok // _WT

    lt, outz = pl.pallas_call(
        _logits_kernel,
        grid=(nb,),
        in_specs=[
            pl.BlockSpec((_BT, hidden), lambda i: (i, 0)),
            pl.BlockSpec((n_exp, hidden), lambda i: (0, 0)),
        ],
        out_specs=[
            pl.BlockSpec((n_exp, _BT), lambda i: (0, i)),
            pl.BlockSpec(memory_space=pltpu.MemorySpace.HBM),
        ],
        out_shape=[
            jax.ShapeDtypeStruct((n_exp, n_tok), jnp.float32),
            jax.ShapeDtypeStruct((n_tok, hidden), jnp.float32),
        ],
        scratch_shapes=[
            pltpu.VMEM((_BT, hidden), jnp.float32),
            pltpu.SemaphoreType.DMA,
        ],
        compiler_params=pltpu.CompilerParams(
            dimension_semantics=("arbitrary",)),
    )(x, gate_w)

    mesh = plsc.VectorSubcoreMesh(core_axis_name="c", subcore_axis_name="s")
    wt, cnts = pl.kernel(
        _sc_route,
        mesh=mesh,
        out_type=[
            jax.ShapeDtypeStruct((n_exp, n_tok), jnp.float32),
            jax.ShapeDtypeStruct((nw, n_exp * _SL), jnp.int32),
        ],
        scratch_types=[
            pltpu.VMEM((n_exp, _WT), jnp.float32),
            pltpu.VMEM((n_exp, _WT), jnp.float32),
            pltpu.VMEM((n_exp * _SL,), jnp.int32),
            pltpu.SemaphoreType.DMA,
        ],
    )(lt)

    out = pl.pallas_call(
        _ffn_kernel,
        in_specs=[
            pl.BlockSpec(memory_space=pltpu.MemorySpace.VMEM),
            pl.BlockSpec(memory_space=pltpu.MemorySpace.HBM),
            pl.BlockSpec(memory_space=pltpu.MemorySpace.HBM),
            pl.BlockSpec(memory_space=pltpu.MemorySpace.HBM),
            pl.BlockSpec(memory_space=pltpu.MemorySpace.HBM),
            pl.BlockSpec(memory_space=pltpu.MemorySpace.HBM),
            pl.BlockSpec(memory_space=pltpu.MemorySpace.HBM),
            pl.BlockSpec(memory_space=pltpu.MemorySpace.HBM),
        ],
        out_specs=pl.BlockSpec(memory_space=pltpu.MemorySpace.HBM),
        out_shape=jax.ShapeDtypeStruct((n_tok, hidden), jnp.float32),
        input_output_aliases={1: 0},
        scratch_shapes=[
            pltpu.SMEM((n_exp + 1,), jnp.int32),
            pltpu.VMEM((n_tok, hidden), jnp.float32),
            pltpu.VMEM((_BT, hidden), jnp.float32),
            pltpu.VMEM((n_exp, _BT), jnp.float32),
            pltpu.VMEM((hidden, hidden), jnp.float32),
            pltpu.VMEM((hidden, hidden), jnp.float32),
            pltpu.VMEM((1, hidden), jnp.float32),
            pltpu.VMEM((1, hidden), jnp.float32),
            pltpu.SemaphoreType.DMA,
            pltpu.SemaphoreType.DMA,
            pltpu.SemaphoreType.DMA,
            pltpu.SemaphoreType.DMA,
            pltpu.SemaphoreType.DMA,
            pltpu.SemaphoreType.DMA,
            pltpu.SemaphoreType.DMA,
        ],
    )(cnts, outz, x, wt, w1, b1.reshape(n_exp, 1, hidden),
      w2, b2.reshape(n_exp, 1, hidden))

    return out.reshape(batch, seq, hidden)
